# baseline (speedup 1.0000x reference)
import jax
import jax.numpy as jnp
from jax.experimental import pallas as pl

_EPS = 1e-5


def _affine_matmul_kernel(h_ref, w_ref, a_ref, c_ref, z_ref):
    h = h_ref[...] * a_ref[...] + c_ref[...]
    z_ref[...] = jnp.dot(h, w_ref[...], preferred_element_type=jnp.float32)


def _affine_matmul(h, w, a, c, bm=1024):
    n, din = h.shape
    dout = w.shape[1]
    return pl.pallas_call(
        _affine_matmul_kernel,
        grid=(n // bm,),
        in_specs=[
            pl.BlockSpec((bm, din), lambda i: (i, 0)),
            pl.BlockSpec((din, dout), lambda i: (0, 0)),
            pl.BlockSpec((1, din), lambda i: (0, 0)),
            pl.BlockSpec((1, din), lambda i: (0, 0)),
        ],
        out_specs=pl.BlockSpec((bm, dout), lambda i: (i, 0)),
        out_shape=jax.ShapeDtypeStruct((n, dout), jnp.float32),
    )(h, w, a, c)


def _agg_kernel(adj_ref, z_ref, b_ref, y_ref, s1_ref, s2_ref):
    i = pl.program_id(0)
    y = jnp.dot(adj_ref[...], z_ref[...], preferred_element_type=jnp.float32)
    y = jnp.maximum(y + b_ref[...], 0.0)
    y_ref[...] = y
    s1 = jnp.sum(y, axis=0, keepdims=True)
    s2 = jnp.sum(y * y, axis=0, keepdims=True)

    @pl.when(i == 0)
    def _init():
        s1_ref[...] = s1
        s2_ref[...] = s2

    @pl.when(i != 0)
    def _acc():
        s1_ref[...] += s1
        s2_ref[...] += s2


def _agg(adj, z, b, bm=512):
    n = adj.shape[0]
    dout = z.shape[1]
    return pl.pallas_call(
        _agg_kernel,
        grid=(n // bm,),
        in_specs=[
            pl.BlockSpec((bm, n), lambda i: (i, 0)),
            pl.BlockSpec((n, dout), lambda i: (0, 0)),
            pl.BlockSpec((1, dout), lambda i: (0, 0)),
        ],
        out_specs=[
            pl.BlockSpec((bm, dout), lambda i: (i, 0)),
            pl.BlockSpec((1, dout), lambda i: (0, 0)),
            pl.BlockSpec((1, dout), lambda i: (0, 0)),
        ],
        out_shape=[
            jax.ShapeDtypeStruct((n, dout), jnp.float32),
            jax.ShapeDtypeStruct((1, dout), jnp.float32),
            jax.ShapeDtypeStruct((1, dout), jnp.float32),
        ],
    )(adj, z, b)


def _affine_kernel(y_ref, a_ref, c_ref, o_ref):
    o_ref[...] = y_ref[...] * a_ref[...] + c_ref[...]


def _affine(y, a, c, bm=1024):
    n, dout = y.shape
    return pl.pallas_call(
        _affine_kernel,
        grid=(n // bm,),
        in_specs=[
            pl.BlockSpec((bm, dout), lambda i: (i, 0)),
            pl.BlockSpec((1, dout), lambda i: (0, 0)),
            pl.BlockSpec((1, dout), lambda i: (0, 0)),
        ],
        out_specs=pl.BlockSpec((bm, dout), lambda i: (i, 0)),
        out_shape=jax.ShapeDtypeStruct((n, dout), jnp.float32),
    )(y, a, c)


def _bn_coeffs(s1, s2, g, beta, n):
    mean = s1 / n
    var = s2 / n - mean * mean
    a = g.reshape(1, -1) / jnp.sqrt(var + _EPS)
    c = beta.reshape(1, -1) - mean * a
    return a, c


def kernel(x, adj, W1, b1, W2, b2, W3, b3, g1, beta1, g2, beta2, g3, beta3):
    n = x.shape[-2]
    h = x.reshape(n, -1)
    adj2 = adj.reshape(n, n)

    din = h.shape[1]
    ones = jnp.ones((1, din), jnp.float32)
    zeros = jnp.zeros((1, din), jnp.float32)

    z = _affine_matmul(h, W1, ones, zeros)
    y, s1, s2 = _agg(adj2, z, b1.reshape(1, -1))
    a, c = _bn_coeffs(s1, s2, g1, beta1, n)

    z = _affine_matmul(y, W2, a, c)
    y, s1, s2 = _agg(adj2, z, b2.reshape(1, -1))
    a, c = _bn_coeffs(s1, s2, g2, beta2, n)

    z = _affine_matmul(y, W3, a, c)
    y, s1, s2 = _agg(adj2, z, b3.reshape(1, -1))
    a, c = _bn_coeffs(s1, s2, g3, beta3, n)

    return _affine(y, a, c)
